# Initial kernel scaffold; baseline (speedup 1.0000x reference)
import jax
import jax.numpy as jnp
from jax.experimental import pallas as pl


def kernel(position_ids, table):
    raise NotImplementedError("write your pallas kernel here")


# speedup vs baseline: 3.3033x; 3.3033x over previous
import functools

import jax
import jax.numpy as jnp
from jax import lax
from jax.experimental import pallas as pl
from jax.experimental.pallas import tpu as pltpu, tpu_sc as plsc

MAX_POS = 8192
EMB = 128

_info = plsc.get_sparse_core_info()
_NC, _NS = _info.num_cores, _info.num_subcores
_NW = _NC * _NS

_B = 4 * 8192
_PER_W = _B // _NW
_GB = 128
_NG = _PER_W // _GB


def _make_kernel():
    mesh = plsc.VectorSubcoreMesh(core_axis_name="c", subcore_axis_name="s")

    @functools.partial(
        pl.kernel,
        mesh=mesh,
        out_type=jax.ShapeDtypeStruct((_B, EMB), jnp.float32),
        scratch_types=[
            pltpu.VMEM((_NG, _GB), jnp.int32),
            pltpu.VMEM((_GB, EMB), jnp.float32),
            pltpu.SemaphoreType.DMA,
        ],
    )
    def gather_kernel(idx_hbm, table_hbm, out_hbm, idx_v, rows_v, sem):
        wid = lax.axis_index("s") * _NC + lax.axis_index("c")
        pltpu.sync_copy(idx_hbm.at[pl.ds(wid * _NG, _NG)], idx_v)
        for j in range(_NG):
            base = wid * _PER_W + j * _GB
            pltpu.async_copy(table_hbm.at[idx_v.at[j]], rows_v, sem).wait()
            pltpu.sync_copy(rows_v, out_hbm.at[pl.ds(base, _GB)])

    return gather_kernel


_gather = _make_kernel()


def kernel(position_ids, table):
    idx = position_ids.reshape(_B // _GB, _GB).astype(jnp.int32)
    out = _gather(idx, table)
    return out.reshape(position_ids.shape + (EMB,))


# speedup vs baseline: 3.7731x; 1.1422x over previous
import functools

import jax
import jax.numpy as jnp
from jax import lax
from jax.experimental import pallas as pl
from jax.experimental.pallas import tpu as pltpu, tpu_sc as plsc

MAX_POS = 8192
EMB = 128

_info = plsc.get_sparse_core_info()
_NC, _NS = _info.num_cores, _info.num_subcores
_NW = _NC * _NS

_B = 4 * 8192
_PER_W = _B // _NW
_GB = 128
_NG = _PER_W // _GB


_K = 4


def _make_kernel():
    mesh = plsc.VectorSubcoreMesh(core_axis_name="c", subcore_axis_name="s")

    @functools.partial(
        pl.kernel,
        mesh=mesh,
        out_type=jax.ShapeDtypeStruct((_B, EMB), jnp.float32),
        scratch_types=[
            pltpu.VMEM((_NG, _GB), jnp.int32),
        ]
        + [pltpu.VMEM((_GB, EMB), jnp.float32) for _ in range(_K)]
        + [pltpu.SemaphoreType.DMA for _ in range(2 * _K)],
    )
    def gather_kernel(idx_hbm, table_hbm, out_hbm, idx_v, *bufs_and_sems):
        bufs = bufs_and_sems[:_K]
        gsems = bufs_and_sems[_K : 2 * _K]
        wsems = bufs_and_sems[2 * _K : 3 * _K]
        wid = lax.axis_index("s") * _NC + lax.axis_index("c")
        pltpu.sync_copy(idx_hbm.at[pl.ds(wid * _NG, _NG)], idx_v)

        def start_gather(j):
            b = j % _K
            return pltpu.async_copy(table_hbm.at[idx_v.at[j]], bufs[b], gsems[b])

        def start_write(j):
            b = j % _K
            base = wid * _PER_W + j * _GB
            return pltpu.async_copy(bufs[b], out_hbm.at[pl.ds(base, _GB)], wsems[b])

        gathers = {j: start_gather(j) for j in range(_K)}
        writes = {}
        for j in range(_NG):
            gathers.pop(j).wait()
            writes[j] = start_write(j)
            if j + _K < _NG:
                writes.pop(j).wait()
                gathers[j + _K] = start_gather(j + _K)
        for j in sorted(writes):
            writes.pop(j).wait()

    return gather_kernel


_gather = _make_kernel()


def kernel(position_ids, table):
    idx = position_ids.reshape(_B // _GB, _GB).astype(jnp.int32)
    out = _gather(idx, table)
    return out.reshape(position_ids.shape + (EMB,))


# speedup vs baseline: 3.8634x; 1.0239x over previous
import functools

import jax
import jax.numpy as jnp
from jax import lax
from jax.experimental import pallas as pl
from jax.experimental.pallas import tpu as pltpu, tpu_sc as plsc

MAX_POS = 8192
EMB = 128

_info = plsc.get_sparse_core_info()
_NC, _NS = _info.num_cores, _info.num_subcores
_NW = _NC * _NS

_B = 4 * 8192
_PER_W = _B // _NW
_GB = 128
_NG = _PER_W // _GB


_K = 6


def _make_kernel():
    mesh = plsc.VectorSubcoreMesh(core_axis_name="c", subcore_axis_name="s")

    @functools.partial(
        pl.kernel,
        mesh=mesh,
        out_type=jax.ShapeDtypeStruct((_B, EMB), jnp.float32),
        scratch_types=[
            pltpu.VMEM((_NG, _GB), jnp.int32),
        ]
        + [pltpu.VMEM((_GB, EMB), jnp.float32) for _ in range(_K)]
        + [pltpu.SemaphoreType.DMA for _ in range(2 * _K)],
    )
    def gather_kernel(idx_hbm, table_hbm, out_hbm, idx_v, *bufs_and_sems):
        bufs = bufs_and_sems[:_K]
        gsems = bufs_and_sems[_K : 2 * _K]
        wsems = bufs_and_sems[2 * _K : 3 * _K]
        wid = lax.axis_index("s") * _NC + lax.axis_index("c")
        pltpu.sync_copy(idx_hbm.at[pl.ds(wid * _NG, _NG)], idx_v)

        def start_gather(j):
            b = j % _K
            return pltpu.async_copy(table_hbm.at[idx_v.at[j]], bufs[b], gsems[b])

        def start_write(j):
            b = j % _K
            base = wid * _PER_W + j * _GB
            return pltpu.async_copy(bufs[b], out_hbm.at[pl.ds(base, _GB)], wsems[b])

        gathers = {j: start_gather(j) for j in range(_K)}
        writes = {}
        for j in range(_NG):
            gathers.pop(j).wait()
            if j - 1 in writes and j - 1 + _K < _NG:
                writes.pop(j - 1).wait()
                gathers[j - 1 + _K] = start_gather(j - 1 + _K)
            writes[j] = start_write(j)
        for j in sorted(writes):
            writes.pop(j).wait()

    return gather_kernel


_gather = _make_kernel()


def kernel(position_ids, table):
    idx = position_ids.reshape(_B // _GB, _GB).astype(jnp.int32)
    out = _gather(idx, table)
    return out.reshape(position_ids.shape + (EMB,))
